# Initial kernel scaffold; baseline (speedup 1.0000x reference)
import jax
import jax.numpy as jnp
from jax.experimental import pallas as pl


def kernel(node_emb, pre_W, pre_b, ln1_g, ln1_b, weight_bases, comp_coeffs, attn_vec, bias, fn_g, fn_b, rows, cols, edge_w, target_node_indices):
    raise NotImplementedError("write your pallas kernel here")


# speedup vs baseline: 6.8137x; 6.8137x over previous
import functools
import jax
import jax.numpy as jnp
from jax import lax
from jax.experimental import pallas as pl
from jax.experimental.pallas import tpu as pltpu
from jax.experimental.pallas import tpu_sc as plsc

N = 10000
D = 128
R = 6
NB = 4
E = 320000
HOPS = 2
T = 8192

NP_ = 10240
NTILE = 16
NCORE = 2
CHUNK = 128
NCH = 160
BB = 16
NBATCH = NCH // BB
ZROWS = 16
EPT = NCH * CHUNK
EP = EPT * NTILE
DEG_R = 128
DEG_C = 80
BN1 = 400


def _k1_body(coeffs_ref, emb_ref, preW_ref, preb_ref, g_ref, b_ref, bases_ref,
             out_ref):
    r = pl.program_id(0)
    x = jnp.dot(emb_ref[...], preW_ref[...].T,
                preferred_element_type=jnp.float32) + preb_ref[...]
    m = jnp.mean(x, axis=-1, keepdims=True)
    v = jnp.mean((x - m) ** 2, axis=-1, keepdims=True)
    x = (x - m) / jnp.sqrt(v + 1e-5) * g_ref[...] + b_ref[...]
    acc = jnp.zeros((BN1, D), jnp.float32)
    for bidx in range(NB):
        w = bases_ref[bidx]
        acc = acc + coeffs_ref[r, bidx] * jnp.dot(
            x, w.T, preferred_element_type=jnp.float32)
    out_ref[0] = acc


def _run_k1(node_emb, pre_W, pre_b, ln1_g, ln1_b, weight_bases, comp_coeffs):
    grid = (R, N // BN1)
    return pl.pallas_call(
        _k1_body,
        grid_spec=pltpu.PrefetchScalarGridSpec(
            num_scalar_prefetch=0,
            grid=grid,
            in_specs=[
                pl.BlockSpec(memory_space=pltpu.SMEM),
                pl.BlockSpec((BN1, D), lambda r, i: (i, 0)),
                pl.BlockSpec((D, D), lambda r, i: (0, 0)),
                pl.BlockSpec((1, D), lambda r, i: (0, 0)),
                pl.BlockSpec((1, D), lambda r, i: (0, 0)),
                pl.BlockSpec((1, D), lambda r, i: (0, 0)),
                pl.BlockSpec((NB, D, D), lambda r, i: (0, 0, 0)),
            ],
            out_specs=pl.BlockSpec((1, BN1, D), lambda r, i: (r, i, 0)),
        ),
        out_shape=jax.ShapeDtypeStruct((R, NP_, D), jnp.float32),
    )(comp_coeffs, node_emb, pre_W, pre_b.reshape(1, D), ln1_g.reshape(1, D),
      ln1_b.reshape(1, D), weight_bases)


def _rsqrt_nr(d16):
    xb = plsc.bitcast(d16, jnp.int32)
    y = plsc.bitcast(jnp.int32(0x5F3759DF) - (xb >> 1), jnp.float32)
    half = d16 * 0.5
    for _ in range(3):
        y = y * (1.5 - half * y * y)
    return jnp.where(d16 > 0.0, y, 0.0)


def _k2_body(xr, rows_p, cols_p, w_p, msgs, t1buf,
             rows_b, cols_b, val_b, degdinv, rowbuf, zbuf, iota128,
             acc, shdeg, sem):
    c = lax.axis_index("c")
    s = lax.axis_index("s")

    z16 = jnp.zeros((16,), jnp.float32)
    for g in range(8):
        iota128[pl.ds(g * 16, 16)] = lax.iota(jnp.int32, 16) + g * 16

    def _zero_zbuf(rr, _):
        for q in range(8):
            zbuf[rr, pl.ds(q * 16, 16)] = z16
        return 0
    lax.fori_loop(0, ZROWS, _zero_zbuf, 0)

    def _load_batch(rel, b):
        pltpu.sync_copy(rows_p.at[rel, s, pl.ds(b * BB, BB)], rows_b)
        pltpu.sync_copy(cols_p.at[rel, s, pl.ds(b * BB, BB)], cols_b)
        pltpu.sync_copy(w_p.at[rel, s, pl.ds(b * BB, BB)], val_b)

    for rloc in range(3):
        rel = c * 3 + rloc

        def _zero_deg(i, _):
            degdinv[pl.ds(i * 16, 16)] = z16
            return 0
        lax.fori_loop(0, NP_ // 16, _zero_deg, 0)

        def _deg_batch(b, _):
            _load_batch(rel, b)

            def _deg_chunk(j, _):
                for q in range(8):
                    cidx = cols_b[j, pl.ds(q * 16, 16)]
                    w16 = val_b[j, pl.ds(q * 16, 16)]
                    plsc.addupdate_scatter(degdinv, [cidx], w16)
                return 0
            lax.fori_loop(0, BB, _deg_chunk, 0)
            return 0
        lax.fori_loop(0, NBATCH, _deg_batch, 0)

        def _pack_deg(rr, _):
            for k in range(5):
                rowbuf[rr, pl.ds(k * 16, 16)] = (
                    degdinv[pl.ds(rr * DEG_C + k * 16, 16)])
            for k in range(5, 8):
                rowbuf[rr, pl.ds(k * 16, 16)] = z16
            return 0
        lax.fori_loop(0, DEG_R, _pack_deg, 0)

        pltpu.sync_copy(zbuf.at[pl.ds(0, 8)], shdeg.at[pl.ds(s * 8, 8)])
        plsc.subcore_barrier()
        pltpu.sync_copy(rowbuf, shdeg.at[iota128], add=True)
        plsc.subcore_barrier()
        pltpu.sync_copy(shdeg, rowbuf)

        def _dinv_row(rr, _):
            for k in range(5):
                d16 = rowbuf[rr, pl.ds(k * 16, 16)]
                degdinv[pl.ds(rr * DEG_C + k * 16, 16)] = _rsqrt_nr(d16)
            return 0
        lax.fori_loop(0, DEG_R, _dinv_row, 0)

        for hop in range(HOPS):
            def _zero_acc(k, _):
                pltpu.sync_copy(zbuf, acc.at[pl.ds(s * 640 + k * ZROWS, ZROWS)])
                return 0
            lax.fori_loop(0, 640 // ZROWS, _zero_acc, 0)
            plsc.subcore_barrier()

            if hop == 0:
                src = xr.at[rel]
            else:
                src = t1buf.at[c]

            def _hop_batch(b, _):
                _load_batch(rel, b)

                def _val_chunk(j, _):
                    for q in range(8):
                        r16 = rows_b[j, pl.ds(q * 16, 16)]
                        c16 = cols_b[j, pl.ds(q * 16, 16)]
                        w16 = val_b[j, pl.ds(q * 16, 16)]
                        dr = plsc.load_gather(degdinv, [r16])
                        dc = plsc.load_gather(degdinv, [c16])
                        val_b[j, pl.ds(q * 16, 16)] = dr * w16 * dc
                    return 0
                lax.fori_loop(0, BB, _val_chunk, 0)

                def _edge_chunk(j, _):
                    pltpu.async_copy(src.at[cols_b.at[j]], rowbuf, sem).wait()

                    def _scale_grp(g, _):
                        val16 = val_b[j, pl.ds(g * 16, 16)]
                        for e in range(16):
                            vb = jnp.full((16,), val16[e], jnp.float32)
                            erow = g * 16 + e
                            for q in range(8):
                                rowbuf[erow, pl.ds(q * 16, 16)] = (
                                    rowbuf[erow, pl.ds(q * 16, 16)] * vb)
                        return 0
                    lax.fori_loop(0, CHUNK // 16, _scale_grp, 0)

                    pltpu.sync_copy(rowbuf, acc.at[rows_b.at[j]], add=True)
                    return 0
                lax.fori_loop(0, BB, _edge_chunk, 0)
                return 0
            lax.fori_loop(0, NBATCH, _hop_batch, 0)
            plsc.subcore_barrier()

            if hop == 0:
                dst = t1buf.at[c]
            else:
                dst = msgs.at[rel]

            def _drain(k, _):
                off = s * 640 + k * 64
                pltpu.sync_copy(acc.at[pl.ds(off, 64)],
                                dst.at[pl.ds(off, 64)])
                return 0
            lax.fori_loop(0, 10, _drain, 0)
            plsc.subcore_barrier()


def _run_k2(xr, rows, cols, edge_w):
    pad = EP - E
    rows_p = jnp.pad(rows, ((0, 0), (0, pad))).reshape(R, NTILE, NCH, CHUNK)
    cols_p = jnp.pad(cols, ((0, 0), (0, pad))).reshape(R, NTILE, NCH, CHUNK)
    w_p = jnp.pad(edge_w, ((0, 0), (0, pad))).reshape(R, NTILE, NCH, CHUNK)

    mesh = plsc.VectorSubcoreMesh(core_axis_name="c", subcore_axis_name="s")
    kfn = pl.kernel(
        _k2_body,
        out_type=[
            jax.ShapeDtypeStruct((R, NP_, D), jnp.float32),
            jax.ShapeDtypeStruct((NCORE, NP_, D), jnp.float32),
        ],
        mesh=mesh,
        compiler_params=pltpu.CompilerParams(needs_layout_passes=False),
        scratch_types=[
            pltpu.VMEM((BB, CHUNK), jnp.int32),
            pltpu.VMEM((BB, CHUNK), jnp.int32),
            pltpu.VMEM((BB, CHUNK), jnp.float32),
            pltpu.VMEM((NP_,), jnp.float32),
            pltpu.VMEM((CHUNK, D), jnp.float32),
            pltpu.VMEM((ZROWS, D), jnp.float32),
            pltpu.VMEM((128,), jnp.int32),
            pltpu.VMEM_SHARED((NP_, D), jnp.float32),
            pltpu.VMEM_SHARED((DEG_R, D), jnp.float32),
            pltpu.SemaphoreType.DMA,
        ],
    )
    msgs, _ = kfn(xr, rows_p, cols_p, w_p)
    return msgs


def _k3_body(msgs_ref, attn_ref, bias_ref, fng_ref, fnb_ref,
             final_ref, aw_ref):
    stacked = [msgs_ref[r] for r in range(R)]
    scores = []
    masks = []
    for r in range(R):
        sr = stacked[r]
        scores.append(jnp.sum(jnp.tanh(sr) * attn_ref[...], axis=1,
                              keepdims=True))
        masks.append(jnp.max(jnp.abs(sr), axis=1, keepdims=True) > 0.0)
    neg = jnp.float32(-3.0e38)
    mscores = [jnp.where(masks[r], scores[r], neg) for r in range(R)]
    m = mscores[0]
    for r in range(1, R):
        m = jnp.maximum(m, mscores[r])
    safe_m = jnp.where(m > neg, m, 0.0)
    es = [jnp.where(masks[r], jnp.exp(scores[r] - safe_m), 0.0)
          for r in range(R)]
    ssum = es[0]
    for r in range(1, R):
        ssum = ssum + es[r]
    safe_s = jnp.where(ssum > 0.0, ssum, 1.0)
    aws = [jnp.where(ssum > 0.0, es[r] / safe_s, 0.0) for r in range(R)]
    out = aws[0] * stacked[0]
    for r in range(1, R):
        out = out + aws[r] * stacked[r]
    out = out + bias_ref[...]
    out = 0.5 * out * (1.0 + lax.erf(out * 0.7071067811865476))
    mm = jnp.mean(out, axis=-1, keepdims=True)
    vv = jnp.mean((out - mm) ** 2, axis=-1, keepdims=True)
    final_ref[...] = (out - mm) / jnp.sqrt(vv + 1e-5) * fng_ref[...] + fnb_ref[...]
    aw_ref[...] = jnp.concatenate(aws, axis=1)


def _run_k3(msgs, attn_vec, bias, fn_g, fn_b):
    grid = (N // BN1,)
    return pl.pallas_call(
        _k3_body,
        grid=grid,
        in_specs=[
            pl.BlockSpec((R, BN1, D), lambda i: (0, i, 0)),
            pl.BlockSpec((1, D), lambda i: (0, 0)),
            pl.BlockSpec((1, D), lambda i: (0, 0)),
            pl.BlockSpec((1, D), lambda i: (0, 0)),
            pl.BlockSpec((1, D), lambda i: (0, 0)),
        ],
        out_specs=[
            pl.BlockSpec((BN1, D), lambda i: (i, 0)),
            pl.BlockSpec((BN1, R), lambda i: (i, 0)),
        ],
        out_shape=[
            jax.ShapeDtypeStruct((N, D), jnp.float32),
            jax.ShapeDtypeStruct((N, R), jnp.float32),
        ],
    )(msgs, attn_vec, bias.reshape(1, D), fn_g.reshape(1, D),
      fn_b.reshape(1, D))


def _k4_body(final_hbm, idx_hbm, out_hbm, idx_v, rows_v, sem):
    wid = lax.axis_index("s") * NCORE + lax.axis_index("c")
    bpw = T // (NCORE * NTILE)
    base = wid * bpw
    pltpu.sync_copy(idx_hbm.at[pl.ds(base, bpw)], idx_v)
    pltpu.async_copy(final_hbm.at[idx_v], rows_v, sem).wait()
    pltpu.sync_copy(rows_v, out_hbm.at[pl.ds(base, bpw)])


def _run_k4(final, target_node_indices):
    bpw = T // (NCORE * NTILE)
    mesh = plsc.VectorSubcoreMesh(core_axis_name="c", subcore_axis_name="s")
    kfn = pl.kernel(
        _k4_body,
        out_type=jax.ShapeDtypeStruct((T, D), jnp.float32),
        mesh=mesh,
        compiler_params=pltpu.CompilerParams(needs_layout_passes=False),
        scratch_types=[
            pltpu.VMEM((bpw,), jnp.int32),
            pltpu.VMEM((bpw, D), jnp.float32),
            pltpu.SemaphoreType.DMA,
        ],
    )
    return kfn(final, target_node_indices)


def kernel(node_emb, pre_W, pre_b, ln1_g, ln1_b, weight_bases, comp_coeffs,
           attn_vec, bias, fn_g, fn_b, rows, cols, edge_w,
           target_node_indices):
    xr = _run_k1(node_emb, pre_W, pre_b, ln1_g, ln1_b, weight_bases,
                 comp_coeffs)
    msgs = _run_k2(xr, rows, cols, edge_w)
    final, aw = _run_k3(msgs, attn_vec, bias, fn_g, fn_b)
    out_t = _run_k4(final, target_node_indices)
    return out_t, aw, node_emb
